# baseline (speedup 1.0000x reference)
import functools

import jax
import jax.numpy as jnp
from jax import lax
from jax.experimental import pallas as pl
from jax.experimental.pallas import tpu as pltpu
from jax.experimental.pallas import tpu_sc as plsc

B, L, D = 1024, 200, 64
N = B * L
NC, NS, LANES = 2, 16, 16
NW = NC * NS
TPW = N // NW
GS = 80
NG = TPW // GS
NB = 4


def _sc_embed(tok_table, tidx3, aidx3, peseg):
    mesh = plsc.VectorSubcoreMesh(core_axis_name="c", subcore_axis_name="s")

    @functools.partial(
        pl.kernel,
        mesh=mesh,
        compiler_params=pltpu.CompilerParams(use_tc_tiling_on_sc=False),
        out_type=jax.ShapeDtypeStruct((N, D), jnp.float32),
        scratch_types=[
            pltpu.VMEM((NG, GS), jnp.int32),
            pltpu.VMEM((NG, GS), jnp.int32),
            pltpu.VMEM((NB, GS, D), jnp.float32),
            pltpu.VMEM((NB, GS, D), jnp.float32),
            pltpu.VMEM((NB, GS, D), jnp.float32),
            [pltpu.SemaphoreType.DMA] * NB,
            [pltpu.SemaphoreType.DMA] * NB,
        ],
    )
    def k(tok_hbm, tidx_hbm, aidx_hbm, peseg_hbm, out_hbm,
          tidx_v, aidx_v, tok_v, add_v, res_v, sem_g, sem_o):
        wid = lax.axis_index("s") * NC + lax.axis_index("c")
        pltpu.sync_copy(tidx_hbm.at[wid], tidx_v)
        pltpu.sync_copy(aidx_hbm.at[wid], aidx_v)

        def fire(g, b):
            pltpu.async_copy(tok_hbm.at[tidx_v.at[g]], tok_v.at[b], sem_g[b])
            pltpu.async_copy(peseg_hbm.at[aidx_v.at[g]], add_v.at[b], sem_g[b])

        for b in range(NB):
            fire(b, b)

        def outer(kk, carry):
            for b in range(NB):
                g = kk * NB + b
                pltpu.make_async_copy(tok_hbm.at[tidx_v.at[g]], tok_v.at[b],
                                      sem_g[b]).wait()
                pltpu.make_async_copy(tok_hbm.at[tidx_v.at[g]], add_v.at[b],
                                      sem_g[b]).wait()
                @pl.when(kk > 0)
                def _():
                    pltpu.make_async_copy(res_v.at[b],
                                          out_hbm.at[pl.ds(0, GS)],
                                          sem_o[b]).wait()

                def row(r, c2):
                    for c in range(D // LANES):
                        sl = pl.ds(c * LANES, LANES)
                        res_v[b, r, sl] = tok_v[b, r, sl] + add_v[b, r, sl]
                    return c2

                lax.fori_loop(0, GS, row, 0, unroll=2)

                @pl.when(g + NB < NG)
                def _():
                    fire(g + NB, b)

                pltpu.async_copy(res_v.at[b],
                                 out_hbm.at[pl.ds(wid * TPW + g * GS, GS)],
                                 sem_o[b])
            return carry

        lax.fori_loop(0, NG // NB, outer, 0)

        for b in range(NB):
            pltpu.make_async_copy(res_v.at[b], out_hbm.at[pl.ds(0, GS)],
                                  sem_o[b]).wait()

    return k(tok_table, tidx3, aidx3, peseg)


def kernel(sequence, segment_labels, tok_table, seg_table, pe):
    tidx3 = sequence.astype(jnp.int32).reshape(NW, NG, GS)
    l_pos = jnp.arange(L, dtype=jnp.int32)
    aidx3 = (segment_labels.astype(jnp.int32) * L + l_pos[None, :]).reshape(NW, NG, GS)
    peseg = (seg_table[:, None, :] + pe[0, :L, :][None, :, :]).reshape(3 * L, D)
    out = _sc_embed(tok_table, tidx3, aidx3, peseg)
    return out.reshape(B, L, D)
